# Initial kernel scaffold; baseline (speedup 1.0000x reference)
import jax
import jax.numpy as jnp
from jax.experimental import pallas as pl


def kernel(url_ids, table):
    raise NotImplementedError("write your pallas kernel here")


# speedup vs baseline: 4.0340x; 4.0340x over previous
import functools

import jax
import jax.numpy as jnp
from jax import lax
from jax.experimental import pallas as pl
from jax.experimental.pallas import tpu as pltpu
from jax.experimental.pallas import tpu_sc as plsc

NC, NS, L = 2, 16, 16
NW = NC * NS
D = 64
C = 128


def _tile_body(idx_hbm, table_hbm, out_hbm, idx_v, rows_v, flag_v):
    wid = lax.axis_index("s") * NC + lax.axis_index("c")
    n_rows = idx_hbm.shape[0] // NW
    n_chunks = n_rows // C
    base = wid * n_rows

    pltpu.sync_copy(idx_hbm.at[pl.ds(base, n_rows)], idx_v.at[pl.ds(0, n_rows)])

    @pl.loop(0, n_chunks)
    def _chunk(c):
        pltpu.sync_copy(table_hbm.at[idx_v.at[pl.ds(c * C, C)]], rows_v)

        acc = jnp.zeros((L,), jnp.bool_)
        for g in range(C // L):
            ig = idx_v[pl.ds(c * C + g * L, L)]
            acc = acc | (ig == 0)
        flag_v[...] = jnp.zeros((L,), jnp.int32)
        plsc.store_scatter(
            flag_v, [jnp.zeros((L,), jnp.int32)], jnp.ones((L,), jnp.int32), mask=acc
        )
        haspad = flag_v[...][0]

        @pl.when(haspad > 0)
        def _fix():
            @pl.loop(0, C)
            def _row(r):
                v = idx_v[pl.ds(c * C + r, L)][0]

                @pl.when(v == 0)
                def _zero_row():
                    z = jnp.zeros((L,), jnp.float32)
                    rr = jnp.full((L,), r, jnp.int32)
                    col = lax.iota(jnp.int32, L)
                    for cg in range(D // L):
                        plsc.store_scatter(rows_v, [rr, col + cg * L], z)

        pltpu.sync_copy(rows_v, out_hbm.at[pl.ds(base + c * C, C)])


def kernel(url_ids, table):
    batch, seq = url_ids.shape
    idx = url_ids.reshape(-1).astype(jnp.int32)
    n = idx.shape[0]
    n_rows = n // NW

    mesh = plsc.VectorSubcoreMesh(
        core_axis_name="c", subcore_axis_name="s", num_cores=NC, num_subcores=NS
    )
    f = pl.kernel(
        _tile_body,
        out_type=jax.ShapeDtypeStruct((n, D), jnp.float32),
        mesh=mesh,
        compiler_params=pltpu.CompilerParams(
            needs_layout_passes=False, use_tc_tiling_on_sc=False
        ),
        scratch_types=[
            pltpu.VMEM((n_rows + L,), jnp.int32),
            pltpu.VMEM((C, D), jnp.float32),
            pltpu.VMEM((L,), jnp.int32),
        ],
    )
    out = f(idx, table)
    return out.reshape(batch, seq, D)


# speedup vs baseline: 4.6639x; 1.1561x over previous
import functools

import jax
import jax.numpy as jnp
from jax import lax
from jax.experimental import pallas as pl
from jax.experimental.pallas import tpu as pltpu
from jax.experimental.pallas import tpu_sc as plsc

NC, NS, L = 2, 16, 16
NW = NC * NS
D = 64
C = 128
NBUF = 5
K = 3


def _tile_body(idx_hbm, table_hbm, out_hbm, idx_v, rows_v, flag_v, *sems):
    gsem = sems[:NBUF]
    osem = sems[NBUF:]
    wid = lax.axis_index("s") * NC + lax.axis_index("c")
    n_rows = idx_hbm.shape[0] // NW
    n_chunks = n_rows // C
    base = wid * n_rows

    pltpu.sync_copy(idx_hbm.at[pl.ds(base, n_rows)], idx_v.at[pl.ds(0, n_rows)])

    def gather_copy(c, b):
        return pltpu.make_async_copy(
            table_hbm.at[idx_v.at[pl.ds(c * C, C)]], rows_v.at[b], gsem[b]
        )

    def out_copy(c, b):
        return pltpu.make_async_copy(
            rows_v.at[b], out_hbm.at[pl.ds(base + c * C, C)], osem[b]
        )

    def fixup(c, b):
        acc = jnp.zeros((L,), jnp.bool_)
        for g in range(C // L):
            ig = idx_v[pl.ds(c * C + g * L, L)]
            acc = acc | (ig == 0)
        flag_v[...] = jnp.zeros((L,), jnp.int32)
        plsc.store_scatter(
            flag_v, [jnp.zeros((L,), jnp.int32)], jnp.ones((L,), jnp.int32), mask=acc
        )
        haspad = flag_v[...][0]

        @pl.when(haspad > 0)
        def _fix():
            @pl.loop(0, C)
            def _row(r):
                v = idx_v[pl.ds(c * C + r, L)][0]

                @pl.when(v == 0)
                def _zero_row():
                    z = jnp.zeros((L,), jnp.float32)
                    rr = jnp.full((L,), r, jnp.int32)
                    col = lax.iota(jnp.int32, L)
                    for cg in range(D // L):
                        plsc.store_scatter(rows_v.at[b], [rr, col + cg * L], z)

    def step(c, b, wait_out, fire):
        gather_copy(c, b).wait()
        fixup(c, b)
        out_copy(c, b).start()
        if fire:
            bf = (b + K) % NBUF
            if wait_out:
                out_copy(0, bf).wait()
            gather_copy(c + K, bf).start()

    for cf in range(K):
        gather_copy(cf, cf).start()
    for b in range(NBUF):
        step(b, b, wait_out=(b + K >= NBUF), fire=True)
    @pl.loop(1, n_chunks // NBUF - 1)
    def _grp(o):
        c0 = o * NBUF
        for b in range(NBUF):
            step(c0 + b, b, wait_out=True, fire=True)

    cl = n_chunks - NBUF
    for b in range(NBUF):
        step(cl + b, b, wait_out=True, fire=(cl + b + K < n_chunks))
    for b in range(NBUF):
        out_copy(0, b).wait()


def kernel(url_ids, table):
    batch, seq = url_ids.shape
    idx = url_ids.reshape(-1).astype(jnp.int32)
    n = idx.shape[0]
    n_rows = n // NW

    mesh = plsc.VectorSubcoreMesh(
        core_axis_name="c", subcore_axis_name="s", num_cores=NC, num_subcores=NS
    )
    f = pl.kernel(
        _tile_body,
        out_type=jax.ShapeDtypeStruct((n, D), jnp.float32),
        mesh=mesh,
        compiler_params=pltpu.CompilerParams(
            needs_layout_passes=False, use_tc_tiling_on_sc=False
        ),
        scratch_types=[
            pltpu.VMEM((n_rows + L,), jnp.int32),
            pltpu.VMEM((NBUF, C, D), jnp.float32),
            pltpu.VMEM((L,), jnp.int32),
        ]
        + [pltpu.SemaphoreType.DMA] * (2 * NBUF),
    )
    out = f(idx, table)
    return out.reshape(batch, seq, D)
